# Initial kernel scaffold; baseline (speedup 1.0000x reference)
import jax
import jax.numpy as jnp
from jax.experimental import pallas as pl


def kernel(importance):
    raise NotImplementedError("write your pallas kernel here")


# speedup vs baseline: 46.2481x; 46.2481x over previous
import jax
import jax.numpy as jnp
from jax import lax
from jax.experimental import pallas as pl

_B, _H, _W = 128, 256, 256
_N = _H * _W
_K = int(0.75 * _N)
_ROWS_PER_BLOCK = 8
_GRID = _B // _ROWS_PER_BLOCK


def _body(x_ref, mask_ref, rowsum_ref):
    x = x_ref[...]
    u = lax.bitcast_convert_type(x, jnp.uint32)
    ku = jnp.where(x < 0, ~u, u | jnp.uint32(0x80000000))

    def bs_bit(i, p):
        cand = p | (jnp.uint32(1) << (31 - i).astype(jnp.uint32))
        cnt = jnp.sum((ku >= cand).astype(jnp.int32), axis=1, keepdims=True)
        return jnp.where(cnt >= _K, cand, p)

    p0 = jnp.zeros((_ROWS_PER_BLOCK, 1), jnp.uint32)
    thr = lax.fori_loop(0, 32, bs_bit, p0)

    gt = ku > thr
    eq = ku == thr
    cnt_gt = jnp.sum(gt.astype(jnp.int32), axis=1, keepdims=True)
    r = _K - cnt_gt

    idx = lax.broadcasted_iota(jnp.int32, (_ROWS_PER_BLOCK, _N), 1)

    def bs_idx(i, t):
        cand = t | (1 << (15 - i))
        cnt = jnp.sum((eq & (idx < cand)).astype(jnp.int32), axis=1,
                      keepdims=True)
        return jnp.where(cnt < r, cand, t)

    t0 = jnp.zeros((_ROWS_PER_BLOCK, 1), jnp.int32)
    t = lax.fori_loop(0, 16, bs_idx, t0)

    mask = (gt | (eq & (idx <= t))).astype(jnp.float32)
    mask_ref[...] = mask
    rowsum_ref[...] = jnp.sum(mask, axis=1, keepdims=True)


def kernel(importance):
    flat = importance.reshape(_B, _N)
    mask, rowsum = pl.pallas_call(
        _body,
        grid=(_GRID,),
        in_specs=[pl.BlockSpec((_ROWS_PER_BLOCK, _N), lambda i: (i, 0))],
        out_specs=[
            pl.BlockSpec((_ROWS_PER_BLOCK, _N), lambda i: (i, 0)),
            pl.BlockSpec((_ROWS_PER_BLOCK, 1), lambda i: (i, 0)),
        ],
        out_shape=[
            jax.ShapeDtypeStruct((_B, _N), jnp.float32),
            jax.ShapeDtypeStruct((_B, 1), jnp.float32),
        ],
    )(flat)
    mean = jnp.sum(rowsum) / jnp.float32(_B * _N)
    return (mask.reshape(_B, 1, _H, _W), mean)
